# baseline (speedup 1.0000x reference)
import functools
import math

import jax
import jax.numpy as jnp
from jax import lax
from jax.experimental import pallas as pl
from jax.experimental.pallas import tpu as pltpu
from jax.experimental.pallas import tpu_sc as plsc

D_MODEL = 64
SCALE = math.sqrt(D_MODEL)

_info = plsc.get_sparse_core_info()
NC, NS, L = _info.num_cores, _info.num_subcores, _info.num_lanes
NW = NC * NS

C = 128
NB = 4


@functools.partial(jax.jit, static_argnames=("nch",))
def _emb_lookup(idx2d, lut, *, nch):
    B = NW * nch * C
    mesh = plsc.VectorSubcoreMesh(core_axis_name="c", subcore_axis_name="s")

    @functools.partial(
        pl.kernel,
        mesh=mesh,
        compiler_params=pltpu.CompilerParams(use_tc_tiling_on_sc=False),
        out_type=jax.ShapeDtypeStruct((B, D_MODEL), jnp.float32),
        scratch_types=[
            pltpu.VMEM((nch, C), jnp.int32),
            pltpu.VMEM((NB, C, D_MODEL), jnp.float32),
        ]
        + [pltpu.SemaphoreType.DMA] * (2 * NB),
    )
    def body(idx_hbm, table_hbm, out_hbm, idx_v, rows_v, *sems):
        gsems, wsems = sems[:NB], sems[NB:]
        wid = lax.axis_index("s") * NC + lax.axis_index("c")
        row0 = wid * (nch * C)

        pltpu.sync_copy(idx_hbm.at[pl.ds(wid * nch, nch)], idx_v)

        def start_gather(chunk, b):
            pltpu.make_async_copy(
                table_hbm.at[idx_v.at[chunk]], rows_v.at[b], gsems[b]
            ).start()

        def wait_gather(chunk, b):
            pltpu.make_async_copy(
                table_hbm.at[idx_v.at[chunk]], rows_v.at[b], gsems[b]
            ).wait()

        def start_write(chunk, b):
            pltpu.make_async_copy(
                rows_v.at[b], out_hbm.at[pl.ds(row0 + chunk * C, C)], wsems[b]
            ).start()

        def wait_write(chunk, b):
            pltpu.make_async_copy(
                rows_v.at[b], out_hbm.at[pl.ds(row0 + chunk * C, C)], wsems[b]
            ).wait()

        def scale(b):
            def srow(r, _):
                for j in range(D_MODEL // L):
                    sl = rows_v[b, r, pl.ds(j * L, L)]
                    rows_v[b, r, pl.ds(j * L, L)] = sl * SCALE
                return 0

            lax.fori_loop(0, C, srow, 0, unroll=4)

        for b in range(NB):
            start_gather(b, b)

        def group(t, _):
            gg = t * NB
            for b in range(NB):
                g = gg + b
                wait_gather(g, b)
                scale(b)
                start_write(g, b)
            for b in range(NB):
                g = gg + b

                @pl.when(g + NB < nch)
                def _():
                    wait_write(g, b)
                    start_gather(g + NB, b)

            return 0

        lax.fori_loop(0, nch // NB, group, 0)

        for b in range(NB):
            wait_write(nch - NB + b, b)

    return body(idx2d, lut)


def kernel(x, lut):
    rows, cols = x.shape
    B = rows * cols
    assert B % (NW * C) == 0
    nch = B // (NW * C)
    idx2d = x.reshape(NW * nch, C).astype(jnp.int32)
    out = _emb_lookup(idx2d, lut, nch=nch)
    return out.reshape(rows, cols, D_MODEL)
